# baseline (speedup 1.0000x reference)
import functools

import jax
import jax.numpy as jnp
from jax import lax
from jax.experimental import pallas as pl
from jax.experimental.pallas import tpu as pltpu
from jax.experimental.pallas import tpu_sc as plsc

_THRESH = 0.7
_MIN_KEPT = 100000
_C = 19
_B = 8
_NC, _NS, _L = 2, 16, 16
_NW = _NC * _NS
_CH = 2048
_PB = 512 * 512
_PW = _B * _PB // _NW
_QW = _PB // _PW
_NCHUNK = _PW // _CH
_NG = _CH // _L

_LN2 = 0.6931471805599453


def _vlog(s):
    bits = plsc.bitcast(s, jnp.int32)
    e = (bits >> 23) - 127
    mant = plsc.bitcast((bits & 0x7FFFFF) | 0x3F800000, jnp.float32)
    u = (mant - 1.0) / (mant + 1.0)
    u2 = u * u
    p = 1.0 + u2 * ((1.0 / 3.0) + u2 * ((1.0 / 5.0) + u2 * ((1.0 / 7.0) + u2 * (1.0 / 9.0))))
    return e.astype(jnp.float32) * _LN2 + 2.0 * u * p


def _sc_body(score_hbm, tgt_hbm, thr_hbm, out_hbm,
             buf_a, buf_b, tbuf_a, tbuf_b, thrv, accv,
             sem_a, sem_b, tsem_a, tsem_b):
    wid = lax.axis_index("s") * _NC + lax.axis_index("c")
    b = wid // _QW
    base_col = (wid % _QW) * _PW

    pltpu.sync_copy(thr_hbm, thrv)
    thr = thrv[...]
    lanes = lax.iota(jnp.int32, _L)

    def start(j, buf, tbuf, sem, tsem):
        col = base_col + j * _CH
        pltpu.make_async_copy(score_hbm.at[b, :, pl.ds(col, _CH)], buf, sem).start()
        pltpu.make_async_copy(tgt_hbm.at[b, pl.ds(col, _CH)], tbuf, tsem).start()

    def wait(j, buf, tbuf, sem, tsem):
        col = base_col + j * _CH
        pltpu.make_async_copy(score_hbm.at[b, :, pl.ds(col, _CH)], buf, sem).wait()
        pltpu.make_async_copy(tgt_hbm.at[b, pl.ds(col, _CH)], tbuf, tsem).wait()

    start(0, buf_a, tbuf_a, sem_a, tsem_a)
    start(1, buf_b, tbuf_b, sem_b, tsem_b)

    def chunk(j, buf, tbuf, sem, tsem, accs):
        wait(j, buf, tbuf, sem, tsem)

        def grp(g, accs):
            c_lt, s_lt, c_le = accs
            base = g * _L
            xs = [buf[c, pl.ds(base, _L)] for c in range(_C)]
            m = xs[0]
            for c in range(1, _C):
                m = jnp.maximum(m, xs[c])
            ssum = jnp.exp(xs[0] - m)
            for c in range(1, _C):
                ssum = ssum + jnp.exp(xs[c] - m)
            tv = tbuf[pl.ds(base, _L)]
            x_t = plsc.load_gather(buf, [tv, base + lanes])
            e_t = jnp.exp(x_t - m)
            pg = e_t / ssum
            nll = _vlog(ssum) + (m - x_t)
            one = jnp.ones((_L,), jnp.float32)
            zero = jnp.zeros((_L,), jnp.float32)
            lt = pg < thr
            return (c_lt + jnp.where(lt, one, zero),
                    s_lt + jnp.where(lt, nll, zero),
                    c_le + jnp.where(pg <= thr, one, zero))

        accs = lax.fori_loop(0, _NG, grp, accs)

        @pl.when(j + 2 < _NCHUNK)
        def _():
            start(j + 2, buf, tbuf, sem, tsem)

        return accs

    zero16 = jnp.zeros((_L,), jnp.float32)

    def pair(i, accs):
        accs = chunk(2 * i, buf_a, tbuf_a, sem_a, tsem_a, accs)
        accs = chunk(2 * i + 1, buf_b, tbuf_b, sem_b, tsem_b, accs)
        return accs

    accs = lax.fori_loop(0, _NCHUNK // 2, pair, (zero16, zero16, zero16))

    accv[pl.ds(0, _L)] = accs[0]
    accv[pl.ds(_L, _L)] = accs[1]
    accv[pl.ds(2 * _L, _L)] = accs[2]
    pltpu.sync_copy(accv, out_hbm.at[wid])


_sc_stats_kernel = functools.partial(
    pl.kernel,
    out_type=jax.ShapeDtypeStruct((_NW, 3 * _L), jnp.float32),
    mesh=plsc.VectorSubcoreMesh(core_axis_name="c", subcore_axis_name="s",
                                num_cores=_NC, num_subcores=_NS),
    compiler_params=pltpu.CompilerParams(use_tc_tiling_on_sc=False,
                                         needs_layout_passes=False),
    scratch_types=[
        pltpu.VMEM((_C, _CH), jnp.float32),
        pltpu.VMEM((_C, _CH), jnp.float32),
        pltpu.VMEM((_CH,), jnp.int32),
        pltpu.VMEM((_CH,), jnp.int32),
        pltpu.VMEM((_L,), jnp.float32),
        pltpu.VMEM((3 * _L,), jnp.float32),
        pltpu.SemaphoreType.DMA,
        pltpu.SemaphoreType.DMA,
        pltpu.SemaphoreType.DMA,
        pltpu.SemaphoreType.DMA,
    ],
)(_sc_body)


def _sc_stats(score, target, thr):
    score3 = score.reshape(_B, _C, _PB)
    tgt2 = target.reshape(_B, _PB)
    thrv = jnp.full((_L,), thr, jnp.float32)
    out = _sc_stats_kernel(score3, tgt2, thrv)
    return (jnp.sum(out[:, 0:_L]),
            jnp.sum(out[:, _L:2 * _L]),
            jnp.sum(out[:, 2 * _L:3 * _L]))


def kernel(score, target):
    kp1 = jnp.float32(_MIN_KEPT + 1)
    cnt_lt, sum_lt, cnt_le = _sc_stats(score, target, jnp.float32(_THRESH))

    def case_a(_):
        return sum_lt / jnp.maximum(cnt_lt, 1.0)

    def case_b(_):
        def cond(st):
            lo, hi = st
            return hi - lo > 1

        def body(st):
            lo, hi = st
            mid = (lo + hi) // 2
            t = lax.bitcast_convert_type(mid, jnp.float32)
            _, _, c_le = _sc_stats(score, target, t)
            ge = c_le >= kp1
            return jnp.where(ge, lo, mid), jnp.where(ge, mid, hi)

        lo0 = jnp.int32(0x3F333333)
        hi0 = jnp.int32(0x3F800000)
        _, hi = lax.while_loop(cond, body, (lo0, hi0))
        vstar = lax.bitcast_convert_type(hi, jnp.float32)
        c_lt2, s_lt2, _ = _sc_stats(score, target, vstar)
        return s_lt2 / jnp.maximum(c_lt2, 1.0)

    return lax.cond(cnt_le < kp1, case_b, case_a, None)
